# baseline (speedup 1.0000x reference)
import functools

import jax
import jax.numpy as jnp
from jax import lax
from jax.experimental import pallas as pl
from jax.experimental.pallas import tpu as pltpu
from jax.experimental.pallas import tpu_sc as plsc

L = 16
CH = 896
NFULL = 34
TAIL = 768
RANGE = 31232
CAP = 768
SEC = 256
NSEC = 3
PIECE = 2048
NPIECE = 8


def kernel(values, table):
    (B,) = values.shape
    V, D = table.shape
    info = plsc.get_sparse_core_info()
    NC, NS = info.num_cores, info.num_subcores
    NW = NC * NS

    table_t = table.T
    table_tail = table[V - 128 :, :].T
    mesh = plsc.VectorSubcoreMesh(core_axis_name="c", subcore_axis_name="s")

    @functools.partial(
        pl.kernel,
        mesh=mesh,
        out_type=jax.ShapeDtypeStruct((B + 64, 128), jnp.float32),
        scratch_types=[
            pltpu.VMEM((PIECE,), jnp.int32),
            pltpu.VMEM((PIECE,), jnp.int32),
            pltpu.VMEM((CAP,), jnp.int32),
            pltpu.VMEM((CAP,), jnp.int32),
            pltpu.VMEM((NSEC, 1, SEC), jnp.int32),
            pltpu.VMEM((D, CH), jnp.float32),
            pltpu.VMEM((D, CH), jnp.float32),
            pltpu.VMEM((CAP * D,), jnp.float32),
            pltpu.VMEM((SEC, 128), jnp.float32),
            pltpu.SemaphoreType.DMA,
            pltpu.SemaphoreType.DMA,
            pltpu.SemaphoreType.DMA,
            pltpu.SemaphoreType.DMA,
        ],
        compiler_params=pltpu.CompilerParams(needs_layout_passes=False),
    )
    def gather_kernel(
        values_hbm, table_hbm, tail_hbm, out_hbm,
        iv0, iv1, lidx, lpos, lpos2, buf0, buf1, ext, pad,
        sem_i, sem0, sem1, sem_s,
    ):
        wid = lax.axis_index("s") * NC + lax.axis_index("c")
        lo = wid * RANGE
        iota = lax.iota(jnp.int32, L)

        is0 = wid == 0
        is1 = wid == 1
        xlo = jnp.where(is0, NW * RANGE, jnp.where(is1, V - 128, 0))
        xhi = jnp.where(is0, V - 64, jnp.where(is1, V, 0))

        for g in range(CAP // L):
            lpos[pl.ds(g * L, L)] = B + ((iota + g * L) & 63)

        pltpu.async_copy(table_hbm.at[:, pl.ds(lo, CH)], buf0, sem0)
        pltpu.async_copy(table_hbm.at[:, pl.ds(lo + CH, CH)], buf1, sem1)
        pltpu.async_copy(values_hbm.at[pl.ds(0, PIECE)], iv0, sem_i)
        cnt = jnp.int32(0)
        for p in range(NPIECE):
            buf = iv0 if p % 2 == 0 else iv1
            pltpu.make_async_copy(
                values_hbm.at[pl.ds(0, PIECE)], iv0, sem_i
            ).wait()
            if p + 1 < NPIECE:
                nbuf = iv1 if p % 2 == 0 else iv0
                pltpu.async_copy(
                    values_hbm.at[pl.ds((p + 1) * PIECE, PIECE)], nbuf, sem_i
                )

            def grp(g, cnt, p=p, buf=buf):
                v = buf[pl.ds(g * L, L)]
                m = ((v >= lo) & (v < lo + RANGE)) | ((v >= xlo) & (v < xhi))
                cn = jnp.minimum(cnt, CAP - L)
                csum = plsc.cumsum(jnp.where(m, 1, 0).astype(jnp.int32))
                pos = cn + csum - 1
                plsc.store_scatter(lidx, [pos], v, mask=m)
                plsc.store_scatter(
                    lpos, [pos], p * PIECE + g * L + iota, mask=m
                )
                return cnt + csum[15]

            cnt = lax.fori_loop(0, PIECE // L, grp, cnt)

        cnt = jnp.minimum(cnt, CAP - L)
        ngr = jnp.minimum((cnt + L - 1) // L, CAP // L)

        def extract(buf, cbase, chw, bk):
            del bk

            def grp(g, _):
                lv = lidx[pl.ds(g * L, L)]
                m = (lv >= cbase) & (lv < cbase + chw)
                hit = plsc.all_reduce_population_count(m)[0]

                @pl.when(hit > 0)
                def _():
                    rel = jnp.clip(lv - cbase, 0, chw - 1)
                    obase = (g * L + iota) * D
                    for c in range(D):
                        vals = plsc.load_gather(
                            buf, [jnp.full((L,), c, jnp.int32), rel]
                        )
                        plsc.store_scatter(ext, [obase + c], vals, mask=m)

                return 0

            lax.fori_loop(0, ngr, grp, 0)

        def chunk_body(k, _):
            cbase = lo + k * CH
            slot = lax.rem(k, 2)

            @pl.when(slot == 0)
            def _():
                pltpu.make_async_copy(
                    table_hbm.at[:, pl.ds(0, CH)], buf0, sem0
                ).wait()
                extract(buf0, cbase, CH, k)

                @pl.when(k + 2 < NFULL)
                def _():
                    pltpu.async_copy(
                        table_hbm.at[:, pl.ds(cbase + 2 * CH, CH)], buf0, sem0
                    )

            @pl.when(slot == 1)
            def _():
                pltpu.make_async_copy(
                    table_hbm.at[:, pl.ds(0, CH)], buf1, sem1
                ).wait()
                extract(buf1, cbase, CH, k)

                @pl.when(k + 2 < NFULL)
                def _():
                    pltpu.async_copy(
                        table_hbm.at[:, pl.ds(cbase + 2 * CH, CH)], buf1, sem1
                    )

            return 0

        lax.fori_loop(0, NFULL, chunk_body, 0)

        tbase = lo + NFULL * CH
        pltpu.sync_copy(
            table_hbm.at[:, pl.ds(tbase, TAIL)], buf0.at[:, pl.ds(0, TAIL)]
        )
        extract(buf0, tbase, TAIL, NFULL)

        @pl.when(is0)
        def _():
            pltpu.sync_copy(
                table_hbm.at[:, pl.ds(NW * RANGE, 512)],
                buf0.at[:, pl.ds(0, 512)],
            )
            extract(buf0, NW * RANGE, 512, NFULL + 1)

        @pl.when(is1)
        def _():
            pltpu.sync_copy(tail_hbm, buf0.at[:, pl.ds(0, 128)])
            extract(buf0, V - 128, 128, NFULL + 1)

        for i in range(CAP // L):
            lpos2[i // (SEC // L), 0, pl.ds((i * L) % SEC, L)] = lpos[
                pl.ds(i * L, L)
            ]

        for s in range(NSEC):

            def padgrp(gg, _, s=s):
                rows = gg * L + iota
                for c in range(D):
                    vals = plsc.load_gather(ext, [(s * SEC + rows) * D + c])
                    plsc.store_scatter(
                        pad, [rows, jnp.full((L,), c, jnp.int32)], vals
                    )
                return 0

            lax.fori_loop(0, SEC // L, padgrp, 0)
            pltpu.async_copy(pad, out_hbm.at[lpos2.at[s, 0]], sem_s)
            pltpu.make_async_copy(pad, out_hbm.at[pl.ds(0, SEC)], sem_s).wait()

    out = gather_kernel(values, table_t, table_tail)
    return out[:B, :D]


# speedup vs baseline: 1.0457x; 1.0457x over previous
import functools

import jax
import jax.numpy as jnp
from jax import lax
from jax.experimental import pallas as pl
from jax.experimental.pallas import tpu as pltpu
from jax.experimental.pallas import tpu_sc as plsc

L = 16
CH = 1024
NFULL = 30
TAIL = 512
RANGE = 31232
CAP = 768
SEC = 128
NSEC = 6
PIECE = 2048
NPIECE = 8


def kernel(values, table):
    (B,) = values.shape
    V, D = table.shape
    info = plsc.get_sparse_core_info()
    NC, NS = info.num_cores, info.num_subcores
    NW = NC * NS

    table_t = table.T
    table_tail = table[V - 128 :, :].T
    mesh = plsc.VectorSubcoreMesh(core_axis_name="c", subcore_axis_name="s")

    @functools.partial(
        pl.kernel,
        mesh=mesh,
        out_type=jax.ShapeDtypeStruct((B + 64, 128), jnp.float32),
        scratch_types=[
            pltpu.VMEM((PIECE,), jnp.int32),
            pltpu.VMEM((PIECE,), jnp.int32),
            pltpu.VMEM((CAP,), jnp.int32),
            pltpu.VMEM((CAP,), jnp.int32),
            pltpu.VMEM((NSEC, 1, SEC), jnp.int32),
            pltpu.VMEM((D, CH), jnp.float32),
            pltpu.VMEM((D, CH), jnp.float32),
            pltpu.VMEM((CAP * D,), jnp.float32),
            pltpu.VMEM((SEC, 128), jnp.float32),
            pltpu.SemaphoreType.DMA,
            pltpu.SemaphoreType.DMA,
            pltpu.SemaphoreType.DMA,
            pltpu.SemaphoreType.DMA,
        ],
        compiler_params=pltpu.CompilerParams(needs_layout_passes=False),
    )
    def gather_kernel(
        values_hbm, table_hbm, tail_hbm, out_hbm,
        iv0, iv1, lidx, lpos, lpos2, buf0, buf1, ext, pad,
        sem_i, sem0, sem1, sem_s,
    ):
        wid = lax.axis_index("s") * NC + lax.axis_index("c")
        lo = wid * RANGE
        iota = lax.iota(jnp.int32, L)

        is0 = wid == 0
        is1 = wid == 1
        xlo = jnp.where(is0, NW * RANGE, jnp.where(is1, V - 128, 0))
        xhi = jnp.where(is0, V - 64, jnp.where(is1, V, 0))

        for g in range(CAP // L):
            lpos[pl.ds(g * L, L)] = B + ((iota + g * L) & 63)

        pltpu.async_copy(table_hbm.at[:, pl.ds(lo, CH)], buf0, sem0)
        pltpu.async_copy(table_hbm.at[:, pl.ds(lo + CH, CH)], buf1, sem1)
        pltpu.async_copy(values_hbm.at[pl.ds(0, PIECE)], iv0, sem_i)
        cnt = jnp.int32(0)
        for p in range(NPIECE):
            buf = iv0 if p % 2 == 0 else iv1
            pltpu.make_async_copy(
                values_hbm.at[pl.ds(0, PIECE)], iv0, sem_i
            ).wait()
            if p + 1 < NPIECE:
                nbuf = iv1 if p % 2 == 0 else iv0
                pltpu.async_copy(
                    values_hbm.at[pl.ds((p + 1) * PIECE, PIECE)], nbuf, sem_i
                )

            def grp(g, cnt, p=p, buf=buf):
                v = buf[pl.ds(g * L, L)]
                m = ((v >= lo) & (v < lo + RANGE)) | ((v >= xlo) & (v < xhi))
                cn = jnp.minimum(cnt, CAP - L)
                csum = plsc.cumsum(jnp.where(m, 1, 0).astype(jnp.int32))
                pos = cn + csum - 1
                plsc.store_scatter(lidx, [pos], v, mask=m)
                plsc.store_scatter(
                    lpos, [pos], p * PIECE + g * L + iota, mask=m
                )
                return cnt + csum[15]

            cnt = lax.fori_loop(0, PIECE // L, grp, cnt)

        cnt = jnp.minimum(cnt, CAP - L)
        ngr = jnp.minimum((cnt + L - 1) // L, CAP // L)

        def extract(buf, cbase, chw, bk):
            del bk

            def grp(g, _):
                lv = lidx[pl.ds(g * L, L)]
                m = (lv >= cbase) & (lv < cbase + chw)
                hit = plsc.all_reduce_population_count(m)[0]

                @pl.when(hit > 0)
                def _():
                    rel = jnp.clip(lv - cbase, 0, chw - 1)
                    ords = g * L + iota
                    for c in range(D):
                        vals = plsc.load_gather(
                            buf, [jnp.full((L,), c, jnp.int32), rel], mask=m
                        )
                        plsc.store_scatter(ext, [ords * D + c], vals, mask=m)

                return 0

            lax.fori_loop(0, ngr, grp, 0)

        def chunk_body(k, _):
            cbase = lo + k * CH
            slot = lax.rem(k, 2)

            @pl.when(slot == 0)
            def _():
                pltpu.make_async_copy(
                    table_hbm.at[:, pl.ds(0, CH)], buf0, sem0
                ).wait()
                extract(buf0, cbase, CH, k)

                @pl.when(k + 2 < NFULL)
                def _():
                    pltpu.async_copy(
                        table_hbm.at[:, pl.ds(cbase + 2 * CH, CH)], buf0, sem0
                    )

            @pl.when(slot == 1)
            def _():
                pltpu.make_async_copy(
                    table_hbm.at[:, pl.ds(0, CH)], buf1, sem1
                ).wait()
                extract(buf1, cbase, CH, k)

                @pl.when(k + 2 < NFULL)
                def _():
                    pltpu.async_copy(
                        table_hbm.at[:, pl.ds(cbase + 2 * CH, CH)], buf1, sem1
                    )

            return 0

        lax.fori_loop(0, NFULL, chunk_body, 0)

        tbase = lo + NFULL * CH
        pltpu.sync_copy(
            table_hbm.at[:, pl.ds(tbase, TAIL)], buf0.at[:, pl.ds(0, TAIL)]
        )
        extract(buf0, tbase, TAIL, NFULL)

        @pl.when(is0)
        def _():
            pltpu.sync_copy(
                table_hbm.at[:, pl.ds(NW * RANGE, 512)],
                buf0.at[:, pl.ds(0, 512)],
            )
            extract(buf0, NW * RANGE, 512, NFULL + 1)

        @pl.when(is1)
        def _():
            pltpu.sync_copy(tail_hbm, buf0.at[:, pl.ds(0, 128)])
            extract(buf0, V - 128, 128, NFULL + 1)

        for i in range(CAP // L):
            lpos2[i // (SEC // L), 0, pl.ds((i * L) % SEC, L)] = lpos[
                pl.ds(i * L, L)
            ]

        for s in range(NSEC):

            def padgrp(gg, _, s=s):
                rows = gg * L + iota
                for c in range(D):
                    vals = plsc.load_gather(ext, [(s * SEC + rows) * D + c])
                    plsc.store_scatter(
                        pad, [rows, jnp.full((L,), c, jnp.int32)], vals
                    )
                return 0

            lax.fori_loop(0, SEC // L, padgrp, 0)
            pltpu.async_copy(pad, out_hbm.at[lpos2.at[s, 0]], sem_s)
            pltpu.make_async_copy(pad, out_hbm.at[pl.ds(0, SEC)], sem_s).wait()

    out = gather_kernel(values, table_t, table_tail)
    return out[:B, :D]
